# Initial kernel scaffold; baseline (speedup 1.0000x reference)
import jax
import jax.numpy as jnp
from jax.experimental import pallas as pl


def kernel(points, batch, enc0_W, enc0_b, enc1_W, enc1_b, dec0_W, dec0_b, dec1_W, dec1_b):
    raise NotImplementedError("write your pallas kernel here")


# speedup vs baseline: 1.0078x; 1.0078x over previous
import jax
import jax.numpy as jnp
import numpy as np
from jax.experimental import pallas as pl

N = 32768
K = 32
R0 = 0.2
R1 = 0.5
M1 = N // K
M2 = M1 // K
D0 = 64
D1 = 128


def _fps(pts, m):
    pts = jax.lax.stop_gradient(pts)
    sel = jnp.zeros((m,), jnp.int32)
    mind = jnp.sum((pts - pts[0]) ** 2, axis=-1)
    def body(i, st):
        sel, mind = st
        nxt = jnp.argmax(mind).astype(jnp.int32)
        sel = sel.at[i].set(nxt)
        mind = jnp.minimum(mind, jnp.sum((pts - pts[nxt]) ** 2, axis=-1))
        return (sel, mind)
    sel, _ = jax.lax.fori_loop(1, m, body, (sel, mind))
    return sel


def _knn_radius(x, y, r, k):
    d2 = jnp.sum(y * y, axis=1)[:, None] + jnp.sum(x * x, axis=1)[None, :] - 2.0 * (y @ x.T)
    neg, idx = jax.lax.top_k(-d2, k)
    valid = (-neg) <= r * r
    return idx, valid


def _decode_kernel(feat_ref, W_ref, b_ref, out2_ref, o_ref):
    d1 = jnp.tanh(feat_ref[...] @ W_ref[...] + b_ref[...])
    o_ref[...] = jnp.tile(out2_ref[...], (1, K)) + d1.reshape(M2 * K, K, 3).reshape(M2 * K, K * 3) * R0


def kernel(points, batch, enc0_W, enc0_b, enc1_W, enc1_b, dec0_W, dec0_b, dec1_W, dec1_b):
    fps1 = _fps(points, M1)
    p1 = points[fps1]
    fps2 = _fps(p1, M2)
    p2 = p1[fps2]
    idx0, valid0 = _knn_radius(points, p1, R0, K)
    rel0 = jnp.where(valid0[..., None], (points[idx0] - p1[:, None, :]) / R0, 0.0)
    h0 = jax.nn.relu(rel0.reshape(-1, 3) @ enc0_W + enc0_b)
    h0 = jnp.where(valid0.reshape(-1, 1), h0, 0.0)
    f1 = h0.reshape(M1, K, D0).max(axis=1)
    idx1, valid1 = _knn_radius(p1, p2, R1, K)
    rel1 = jnp.where(valid1[..., None], (p1[idx1] - p2[:, None, :]) / R1, 0.0)
    g1 = jnp.where(valid1[..., None], f1[idx1], 0.0)
    h1 = jax.nn.relu(jnp.concatenate([rel1, g1], axis=-1).reshape(-1, 3 + D0) @ enc1_W + enc1_b)
    h1 = jnp.where(valid1.reshape(-1, 1), h1, 0.0)
    f2 = h1.reshape(M2, K, D1).max(axis=1)
    cur = idx1.reshape(-1)
    input_points1 = p1[cur]
    nxt = idx0[cur].reshape(-1)
    input_points0 = points[nxt]
    d0 = (f2 @ dec0_W + dec0_b).reshape(M2, K, 3 + D0)
    rel_a = jnp.tanh(d0[..., :3]).reshape(M2 * K, 3)
    feat_a = jax.nn.relu(d0[..., 3:]).reshape(M2 * K, D0)
    out2 = jnp.repeat(p2, K, axis=0) + rel_a * R1
    out3 = pl.pallas_call(
        _decode_kernel,
        out_shape=jax.ShapeDtypeStruct((M2 * K, K * 3), jnp.float32),
    )(feat_a, dec1_W, dec1_b.reshape(1, K * 3), out2).reshape(M2 * K * K, 3)
    return (out3, f2, input_points0, input_points1)


# speedup vs baseline: 1.3528x; 1.3423x over previous
import functools

import jax
import jax.numpy as jnp
import numpy as np
from jax.experimental import pallas as pl
from jax.experimental.pallas import tpu as pltpu

N = 32768
K = 32
R0 = 0.2
R1 = 0.5
M1 = N // K
M2 = M1 // K
D0 = 64
D1 = 128


def _fps_kernel(px_ref, py_ref, pz_ref,
                p1x_ref, p1y_ref, p1z_ref, p2x_ref, p2y_ref, p2z_ref,
                mind_ref):
    rows = jax.lax.broadcasted_iota(jnp.int32, (256, 128), 0)
    cols = jax.lax.broadcasted_iota(jnp.int32, (256, 128), 1)
    iof = rows * 128 + cols
    r1 = jax.lax.broadcasted_iota(jnp.int32, (8, 128), 0)
    c1 = jax.lax.broadcasted_iota(jnp.int32, (8, 128), 1)
    io1 = r1 * 128 + c1

    px = px_ref[...]
    py = py_ref[...]
    pz = pz_ref[...]

    def _sel_coord(arr, oh):
        return jnp.sum(jnp.where(oh, arr, 0.0))

    oh0 = iof == 0
    qx = _sel_coord(px, oh0)
    qy = _sel_coord(py, oh0)
    qz = _sel_coord(pz, oh0)
    dx = px - qx
    dy = py - qy
    dz = pz - qz
    mind_ref[...] = (dx * dx + dy * dy) + dz * dz
    ohw = io1 == 0
    p1x_ref[...] = jnp.where(ohw, qx, 0.0)
    p1y_ref[...] = jnp.where(ohw, qy, 0.0)
    p1z_ref[...] = jnp.where(ohw, qz, 0.0)

    def body(i, _):
        mind = mind_ref[...]
        m = jnp.max(mind)
        nxt = jnp.min(jnp.where(mind == m, iof, (2**30)))
        oh = iof == nxt
        qx = _sel_coord(px, oh)
        qy = _sel_coord(py, oh)
        qz = _sel_coord(pz, oh)
        dx = px - qx
        dy = py - qy
        dz = pz - qz
        d = (dx * dx + dy * dy) + dz * dz
        mind_ref[...] = jnp.minimum(mind, d)
        ohw = io1 == i
        p1x_ref[...] = jnp.where(ohw, qx, p1x_ref[...])
        p1y_ref[...] = jnp.where(ohw, qy, p1y_ref[...])
        p1z_ref[...] = jnp.where(ohw, qz, p1z_ref[...])
        return 0

    jax.lax.fori_loop(1, M1, body, 0)

    gx = p1x_ref[...]
    gy = p1y_ref[...]
    gz = p1z_ref[...]
    oh0b = io1 == 0
    qx = _sel_coord(gx, oh0b)
    qy = _sel_coord(gy, oh0b)
    qz = _sel_coord(gz, oh0b)
    dx = gx - qx
    dy = gy - qy
    dz = gz - qz
    mind2 = (dx * dx + dy * dy) + dz * dz
    p2x_ref[...] = jnp.where(oh0b, qx, 0.0)
    p2y_ref[...] = jnp.where(oh0b, qy, 0.0)
    p2z_ref[...] = jnp.where(oh0b, qz, 0.0)

    def body2(i, mind2):
        m = jnp.max(mind2)
        nxt = jnp.min(jnp.where(mind2 == m, io1, (2**30)))
        oh = io1 == nxt
        qx = _sel_coord(gx, oh)
        qy = _sel_coord(gy, oh)
        qz = _sel_coord(gz, oh)
        dx = gx - qx
        dy = gy - qy
        dz = gz - qz
        d = (dx * dx + dy * dy) + dz * dz
        ohw = io1 == i
        p2x_ref[...] = jnp.where(ohw, qx, p2x_ref[...])
        p2y_ref[...] = jnp.where(ohw, qy, p2y_ref[...])
        p2z_ref[...] = jnp.where(ohw, qz, p2z_ref[...])
        return jnp.minimum(mind2, d)

    jax.lax.fori_loop(1, M2, body2, mind2)


def _fps_pallas(points):
    px = points[:, 0].reshape(256, 128)
    py = points[:, 1].reshape(256, 128)
    pz = points[:, 2].reshape(256, 128)
    shp = jax.ShapeDtypeStruct((8, 128), jnp.float32)
    outs = pl.pallas_call(
        _fps_kernel,
        out_shape=(shp,) * 6,
        scratch_shapes=[pltpu.VMEM((256, 128), jnp.float32)],
    )(px, py, pz)
    p1 = jnp.stack([o.reshape(M1) for o in outs[:3]], axis=1)
    p2 = jnp.stack([o.reshape(M1)[:M2] for o in outs[3:]], axis=1)
    return p1, p2


def _d2_kernel(y_ref, x_ref, o_ref):
    y = y_ref[...]
    x = x_ref[...]
    yy = y[:, 0:1] * y[:, 0:1] + y[:, 1:2] * y[:, 1:2] + y[:, 2:3] * y[:, 2:3]
    xx = x[:, 0:1] * x[:, 0:1] + x[:, 1:2] * x[:, 1:2] + x[:, 2:3] * x[:, 2:3]
    m = jax.lax.dot_general(y, x, (((1,), (1,)), ((), ())),
                            preferred_element_type=jnp.float32)
    o_ref[...] = (yy + xx.T) - 2.0 * m


def _d2_pallas(y, x):
    My, Nx = y.shape[0], x.shape[0]
    QB = min(My, 256)
    CB = min(Nx, 4096)
    return pl.pallas_call(
        _d2_kernel,
        grid=(My // QB, Nx // CB),
        in_specs=[
            pl.BlockSpec((QB, 3), lambda i, j: (i, 0)),
            pl.BlockSpec((CB, 3), lambda i, j: (j, 0)),
        ],
        out_specs=pl.BlockSpec((QB, CB), lambda i, j: (i, j)),
        out_shape=jax.ShapeDtypeStruct((My, Nx), jnp.float32),
    )(y, x)


def _knn_radius(x, y, r, k):
    d2 = (jnp.sum(y * y, axis=1)[:, None] + jnp.sum(x * x, axis=1)[None, :]
          - 2.0 * (y @ x.T))
    neg, idx = jax.lax.top_k(-d2, k)
    valid = (-neg) <= r * r
    return idx, valid


_QB0 = 8


def _knn0_kernel(y_ref, xt_ref, idx_ref, mval_ref, d2_ref, bm_ref):
    y = y_ref[...]
    xt = xt_ref[...]
    yy = y[:, 0:1] * y[:, 0:1] + y[:, 1:2] * y[:, 1:2] + y[:, 2:3] * y[:, 2:3]
    xx = xt[0:1, :] * xt[0:1, :] + xt[1:2, :] * xt[1:2, :] + xt[2:3, :] * xt[2:3, :]
    mm = jnp.dot(y, xt, preferred_element_type=jnp.float32)
    d2 = (yy + xx) - 2.0 * mm
    d2_ref[...] = d2.reshape(_QB0, N // 128, 128)
    bm_ref[...] = jnp.min(d2_ref[...], axis=2)
    io_bm = jax.lax.broadcasted_iota(jnp.int32, (_QB0, N // 128), 1)
    io256 = jax.lax.broadcasted_iota(jnp.int32, (1, N // 128), 1)
    io128 = jax.lax.broadcasted_iota(jnp.int32, (1, 128), 1)
    io32 = jax.lax.broadcasted_iota(jnp.int32, (1, K), 1)

    def step(kk, _):
        bm = bm_ref[...]
        mq = jnp.min(bm, axis=1, keepdims=True)
        rq = jnp.min(jnp.where(bm == mq, io_bm, 2**30), axis=1,
                     keepdims=True)
        for q in range(_QB0):
            ms = mq[q, 0]
            r = rq[q, 0]
            row = d2_ref[q, pl.ds(r, 1), :]
            c = jnp.min(jnp.where(row == ms, io128, 2**30))
            flat = r * 128 + c
            row2 = jnp.where(io128 == c, float("inf"), row)
            d2_ref[q, pl.ds(r, 1), :] = row2
            nm = jnp.min(row2)
            bmrow = bm_ref[pl.ds(q, 1), :]
            bm_ref[pl.ds(q, 1), :] = jnp.where(io256 == r, nm, bmrow)
            idx_ref[pl.ds(q, 1), :] = jnp.where(io32 == kk, flat,
                                                idx_ref[pl.ds(q, 1), :])
            mval_ref[pl.ds(q, 1), :] = jnp.where(io32 == kk, ms,
                                                 mval_ref[pl.ds(q, 1), :])
        return 0

    jax.lax.fori_loop(0, K, step, 0)


def _knn0_pallas(points, p1, r):
    xt = points.T
    idx, mval = pl.pallas_call(
        _knn0_kernel,
        grid=(M1 // _QB0,),
        in_specs=[
            pl.BlockSpec((_QB0, 3), lambda i: (i, 0)),
            pl.BlockSpec((3, N), lambda i: (0, 0)),
        ],
        out_specs=[
            pl.BlockSpec((_QB0, K), lambda i: (i, 0)),
            pl.BlockSpec((_QB0, K), lambda i: (i, 0)),
        ],
        out_shape=[
            jax.ShapeDtypeStruct((M1, K), jnp.int32),
            jax.ShapeDtypeStruct((M1, K), jnp.float32),
        ],
        scratch_shapes=[
            pltpu.VMEM((_QB0, N // 128, 128), jnp.float32),
            pltpu.VMEM((_QB0, N // 128), jnp.float32),
        ],
    )(p1, xt)
    valid = mval <= r * r
    return idx, valid


def _decode_kernel(feat_ref, W_ref, b_ref, out2_ref, o_ref):
    d1 = jnp.tanh(feat_ref[...] @ W_ref[...] + b_ref[...])
    o_ref[...] = jnp.tile(out2_ref[...], (1, K)) + d1 * R0


def kernel(points, batch, enc0_W, enc0_b, enc1_W, enc1_b, dec0_W, dec0_b, dec1_W, dec1_b):
    p1, p2 = _fps_pallas(points)
    idx0, valid0 = _knn0_pallas(points, p1, R0)
    rel0 = jnp.where(valid0[..., None], (points[idx0] - p1[:, None, :]) / R0, 0.0)
    h0 = jax.nn.relu(rel0.reshape(-1, 3) @ enc0_W + enc0_b)
    h0 = jnp.where(valid0.reshape(-1, 1), h0, 0.0)
    f1 = h0.reshape(M1, K, D0).max(axis=1)
    idx1, valid1 = _knn_radius(p1, p2, R1, K)
    rel1 = jnp.where(valid1[..., None], (p1[idx1] - p2[:, None, :]) / R1, 0.0)
    g1 = jnp.where(valid1[..., None], f1[idx1], 0.0)
    h1 = jax.nn.relu(jnp.concatenate([rel1, g1], axis=-1).reshape(-1, 3 + D0) @ enc1_W + enc1_b)
    h1 = jnp.where(valid1.reshape(-1, 1), h1, 0.0)
    f2 = h1.reshape(M2, K, D1).max(axis=1)
    cur = idx1.reshape(-1)
    input_points1 = p1[cur]
    nxt = idx0[cur].reshape(-1)
    input_points0 = points[nxt]
    d0 = (f2 @ dec0_W + dec0_b).reshape(M2, K, 3 + D0)
    rel_a = jnp.tanh(d0[..., :3]).reshape(M2 * K, 3)
    feat_a = jax.nn.relu(d0[..., 3:]).reshape(M2 * K, D0)
    out2 = jnp.repeat(p2, K, axis=0) + rel_a * R1
    out3 = pl.pallas_call(
        _decode_kernel,
        out_shape=jax.ShapeDtypeStruct((M2 * K, K * 3), jnp.float32),
    )(feat_a, dec1_W, dec1_b.reshape(1, K * 3), out2).reshape(M2 * K * K, 3)
    return (out3, f2, input_points0, input_points1)


# speedup vs baseline: 3.8653x; 2.8572x over previous
import functools

import jax
import jax.numpy as jnp
import numpy as np
from jax.experimental import pallas as pl
from jax.experimental.pallas import tpu as pltpu

N = 32768
K = 32
R0 = 0.2
R1 = 0.5
M1 = N // K
M2 = M1 // K
D0 = 64
D1 = 128


def _fps_kernel(px_ref, py_ref, pz_ref,
                p1x_ref, p1y_ref, p1z_ref, p2x_ref, p2y_ref, p2z_ref,
                mind_ref):
    rows = jax.lax.broadcasted_iota(jnp.int32, (256, 128), 0)
    cols = jax.lax.broadcasted_iota(jnp.int32, (256, 128), 1)
    iof = rows * 128 + cols
    r1 = jax.lax.broadcasted_iota(jnp.int32, (8, 128), 0)
    c1 = jax.lax.broadcasted_iota(jnp.int32, (8, 128), 1)
    io1 = r1 * 128 + c1

    px = px_ref[...]
    py = py_ref[...]
    pz = pz_ref[...]

    def _sel_coord(arr, oh):
        return jnp.sum(jnp.where(oh, arr, 0.0))

    oh0 = iof == 0
    qx = _sel_coord(px, oh0)
    qy = _sel_coord(py, oh0)
    qz = _sel_coord(pz, oh0)
    dx = px - qx
    dy = py - qy
    dz = pz - qz
    mind_ref[...] = (dx * dx + dy * dy) + dz * dz
    ohw = io1 == 0
    p1x_ref[...] = jnp.where(ohw, qx, 0.0)
    p1y_ref[...] = jnp.where(ohw, qy, 0.0)
    p1z_ref[...] = jnp.where(ohw, qz, 0.0)

    def body(i, _):
        mind = mind_ref[...]
        m = jnp.max(mind)
        nxt = jnp.min(jnp.where(mind == m, iof, (2**30)))
        oh = iof == nxt
        qx = _sel_coord(px, oh)
        qy = _sel_coord(py, oh)
        qz = _sel_coord(pz, oh)
        dx = px - qx
        dy = py - qy
        dz = pz - qz
        d = (dx * dx + dy * dy) + dz * dz
        mind_ref[...] = jnp.minimum(mind, d)
        ohw = io1 == i
        p1x_ref[...] = jnp.where(ohw, qx, p1x_ref[...])
        p1y_ref[...] = jnp.where(ohw, qy, p1y_ref[...])
        p1z_ref[...] = jnp.where(ohw, qz, p1z_ref[...])
        return 0

    jax.lax.fori_loop(1, M1, body, 0)

    gx = p1x_ref[...]
    gy = p1y_ref[...]
    gz = p1z_ref[...]
    oh0b = io1 == 0
    qx = _sel_coord(gx, oh0b)
    qy = _sel_coord(gy, oh0b)
    qz = _sel_coord(gz, oh0b)
    dx = gx - qx
    dy = gy - qy
    dz = gz - qz
    mind2 = (dx * dx + dy * dy) + dz * dz
    p2x_ref[...] = jnp.where(oh0b, qx, 0.0)
    p2y_ref[...] = jnp.where(oh0b, qy, 0.0)
    p2z_ref[...] = jnp.where(oh0b, qz, 0.0)

    def body2(i, mind2):
        m = jnp.max(mind2)
        nxt = jnp.min(jnp.where(mind2 == m, io1, (2**30)))
        oh = io1 == nxt
        qx = _sel_coord(gx, oh)
        qy = _sel_coord(gy, oh)
        qz = _sel_coord(gz, oh)
        dx = gx - qx
        dy = gy - qy
        dz = gz - qz
        d = (dx * dx + dy * dy) + dz * dz
        ohw = io1 == i
        p2x_ref[...] = jnp.where(ohw, qx, p2x_ref[...])
        p2y_ref[...] = jnp.where(ohw, qy, p2y_ref[...])
        p2z_ref[...] = jnp.where(ohw, qz, p2z_ref[...])
        return jnp.minimum(mind2, d)

    jax.lax.fori_loop(1, M2, body2, mind2)


def _fps_pallas(points):
    px = points[:, 0].reshape(256, 128)
    py = points[:, 1].reshape(256, 128)
    pz = points[:, 2].reshape(256, 128)
    shp = jax.ShapeDtypeStruct((8, 128), jnp.float32)
    outs = pl.pallas_call(
        _fps_kernel,
        out_shape=(shp,) * 6,
        scratch_shapes=[pltpu.VMEM((256, 128), jnp.float32)],
    )(px, py, pz)
    p1 = jnp.stack([o.reshape(M1) for o in outs[:3]], axis=1)
    p2 = jnp.stack([o.reshape(M1)[:M2] for o in outs[3:]], axis=1)
    return p1, p2


def _d2_kernel(y_ref, x_ref, o_ref):
    y = y_ref[...]
    x = x_ref[...]
    yy = y[:, 0:1] * y[:, 0:1] + y[:, 1:2] * y[:, 1:2] + y[:, 2:3] * y[:, 2:3]
    xx = x[:, 0:1] * x[:, 0:1] + x[:, 1:2] * x[:, 1:2] + x[:, 2:3] * x[:, 2:3]
    m = jax.lax.dot_general(y, x, (((1,), (1,)), ((), ())),
                            preferred_element_type=jnp.float32)
    o_ref[...] = (yy + xx.T) - 2.0 * m


def _d2_pallas(y, x):
    My, Nx = y.shape[0], x.shape[0]
    QB = min(My, 256)
    CB = min(Nx, 4096)
    return pl.pallas_call(
        _d2_kernel,
        grid=(My // QB, Nx // CB),
        in_specs=[
            pl.BlockSpec((QB, 3), lambda i, j: (i, 0)),
            pl.BlockSpec((CB, 3), lambda i, j: (j, 0)),
        ],
        out_specs=pl.BlockSpec((QB, CB), lambda i, j: (i, j)),
        out_shape=jax.ShapeDtypeStruct((My, Nx), jnp.float32),
    )(y, x)


def _knn_radius(x, y, r, k):
    d2 = (jnp.sum(y * y, axis=1)[:, None] + jnp.sum(x * x, axis=1)[None, :]
          - 2.0 * (y @ x.T))
    neg, idx = jax.lax.top_k(-d2, k)
    valid = (-neg) <= r * r
    return idx, valid


_QB0 = 8


def _knn0_kernel(y_ref, xt_ref, idx_ref, mval_ref, d2_ref, bm_ref):
    y = y_ref[...]
    xt = xt_ref[...]
    yy = y[:, 0:1] * y[:, 0:1] + y[:, 1:2] * y[:, 1:2] + y[:, 2:3] * y[:, 2:3]
    xx = xt[0:1, :] * xt[0:1, :] + xt[1:2, :] * xt[1:2, :] + xt[2:3, :] * xt[2:3, :]
    mm = jnp.dot(y, xt, preferred_element_type=jnp.float32)
    d2 = (yy + xx) - 2.0 * mm
    d2_ref[...] = d2.reshape(_QB0, N // 128, 128)
    bm_ref[...] = jnp.min(d2_ref[...], axis=2)
    io_bm = jax.lax.broadcasted_iota(jnp.int32, (_QB0, N // 128), 1)
    io256 = jax.lax.broadcasted_iota(jnp.int32, (1, N // 128), 1)
    io128 = jax.lax.broadcasted_iota(jnp.int32, (1, 128), 1)
    io32 = jax.lax.broadcasted_iota(jnp.int32, (1, K), 1)

    def step(kk, _):
        bm = bm_ref[...]
        mq = jnp.min(bm, axis=1, keepdims=True)
        rq = jnp.min(jnp.where(bm == mq, io_bm, 2**30), axis=1,
                     keepdims=True)
        for q in range(_QB0):
            ms = mq[q, 0]
            r = rq[q, 0]
            row = d2_ref[q, pl.ds(r, 1), :]
            c = jnp.min(jnp.where(row == ms, io128, 2**30))
            flat = r * 128 + c
            row2 = jnp.where(io128 == c, float("inf"), row)
            d2_ref[q, pl.ds(r, 1), :] = row2
            nm = jnp.min(row2)
            bmrow = bm_ref[pl.ds(q, 1), :]
            bm_ref[pl.ds(q, 1), :] = jnp.where(io256 == r, nm, bmrow)
            idx_ref[pl.ds(q, 1), :] = jnp.where(io32 == kk, flat,
                                                idx_ref[pl.ds(q, 1), :])
            mval_ref[pl.ds(q, 1), :] = jnp.where(io32 == kk, ms,
                                                 mval_ref[pl.ds(q, 1), :])
        return 0

    jax.lax.fori_loop(0, K, step, 0)


def _knn0_pallas(points, p1, r):
    xt = points.T
    idx, mval = pl.pallas_call(
        _knn0_kernel,
        grid=(M1 // _QB0,),
        in_specs=[
            pl.BlockSpec((_QB0, 3), lambda i: (i, 0)),
            pl.BlockSpec((3, N), lambda i: (0, 0)),
        ],
        out_specs=[
            pl.BlockSpec((_QB0, K), lambda i: (i, 0)),
            pl.BlockSpec((_QB0, K), lambda i: (i, 0)),
        ],
        out_shape=[
            jax.ShapeDtypeStruct((M1, K), jnp.int32),
            jax.ShapeDtypeStruct((M1, K), jnp.float32),
        ],
        scratch_shapes=[
            pltpu.VMEM((_QB0, N // 128, 128), jnp.float32),
            pltpu.VMEM((_QB0, N // 128), jnp.float32),
        ],
    )(p1, xt)
    valid = mval <= r * r
    return idx, valid


def _decode_kernel(feat_ref, W_ref, b_ref, out2_ref, o_ref):
    d1 = jnp.tanh(feat_ref[...] @ W_ref[...] + b_ref[...])
    o_ref[...] = jnp.tile(out2_ref[...], (1, K)) + d1 * R0


def kernel(points, batch, enc0_W, enc0_b, enc1_W, enc1_b, dec0_W, dec0_b, dec1_W, dec1_b):
    p1, p2 = _fps_pallas(points)
    idx0, valid0 = _knn_radius(points, p1, R0, K)
    rel0 = jnp.where(valid0[..., None], (points[idx0] - p1[:, None, :]) / R0, 0.0)
    h0 = jax.nn.relu(rel0.reshape(-1, 3) @ enc0_W + enc0_b)
    h0 = jnp.where(valid0.reshape(-1, 1), h0, 0.0)
    f1 = h0.reshape(M1, K, D0).max(axis=1)
    idx1, valid1 = _knn_radius(p1, p2, R1, K)
    rel1 = jnp.where(valid1[..., None], (p1[idx1] - p2[:, None, :]) / R1, 0.0)
    g1 = jnp.where(valid1[..., None], f1[idx1], 0.0)
    h1 = jax.nn.relu(jnp.concatenate([rel1, g1], axis=-1).reshape(-1, 3 + D0) @ enc1_W + enc1_b)
    h1 = jnp.where(valid1.reshape(-1, 1), h1, 0.0)
    f2 = h1.reshape(M2, K, D1).max(axis=1)
    cur = idx1.reshape(-1)
    input_points1 = p1[cur]
    nxt = idx0[cur].reshape(-1)
    input_points0 = points[nxt]
    d0 = (f2 @ dec0_W + dec0_b).reshape(M2, K, 3 + D0)
    rel_a = jnp.tanh(d0[..., :3]).reshape(M2 * K, 3)
    feat_a = jax.nn.relu(d0[..., 3:]).reshape(M2 * K, D0)
    out2 = jnp.repeat(p2, K, axis=0) + rel_a * R1
    out3 = pl.pallas_call(
        _decode_kernel,
        out_shape=jax.ShapeDtypeStruct((M2 * K, K * 3), jnp.float32),
    )(feat_a, dec1_W, dec1_b.reshape(1, K * 3), out2).reshape(M2 * K * K, 3)
    return (out3, f2, input_points0, input_points1)
